# Initial kernel scaffold; baseline (speedup 1.0000x reference)
import jax
import jax.numpy as jnp
from jax.experimental import pallas as pl


def kernel(x, edge_index, Wl1, bl1, Wr1, Wl2, bl2, Wr2):
    raise NotImplementedError("write your pallas kernel here")


# speedup vs baseline: 8.6956x; 8.6956x over previous
import functools

import jax
import jax.numpy as jnp
from jax import lax
from jax.experimental import pallas as pl
from jax.experimental.pallas import tpu as pltpu
from jax.experimental.pallas import tpu_sc as plsc

NC = 2
NS = 16
NW = NC * NS
CH = 128


def _make_sc_segment_sum(NP, CPW, D, with_count):
  mesh = plsc.VectorSubcoreMesh(core_axis_name="c", subcore_axis_name="s")
  out_type = [jax.ShapeDtypeStruct((NC, NP, D), jnp.float32)]
  scratch = [
      pltpu.VMEM((CPW, CH), jnp.int32),
      pltpu.VMEM((CPW, CH), jnp.int32),
      pltpu.VMEM((CH, D), jnp.float32),
      pltpu.VMEM((CH, D), jnp.float32),
      pltpu.VMEM_SHARED((NP, D), jnp.float32),
      pltpu.SemaphoreType.DMA,
      pltpu.SemaphoreType.DMA,
  ]
  if with_count:
    out_type.append(jax.ShapeDtypeStruct((NC, NP, 16), jnp.float32))
    scratch.append(pltpu.VMEM((CH, 16), jnp.float32))
    scratch.append(pltpu.VMEM_SHARED((NP, 16), jnp.float32))

  rows_per_tile = NP // NS
  zchunks = NP // CH // NS

  def body(feat, srcidx, dstidx, *rest):
    if with_count:
      (out_acc, out_cnt, src_slab, dst_slab, rows0, rows1, acc_sh,
       sem0, sem1, ones_v, cnt_sh) = rest
    else:
      (out_acc, src_slab, dst_slab, rows0, rows1, acc_sh, sem0, sem1) = rest
    c = lax.axis_index("c")
    s = lax.axis_index("s")
    wid = c * NS + s
    zero16 = jnp.zeros((16,), jnp.float32)
    one16 = jnp.ones((16,), jnp.float32)

    def zrow(i, _):
      for d4 in range(D // 16):
        rows0[i, pl.ds(d4 * 16, 16)] = zero16
      return 0
    lax.fori_loop(0, CH, zrow, 0)
    for j in range(zchunks):
      pltpu.sync_copy(rows0, acc_sh.at[pl.ds((s * zchunks + j) * CH, CH)])
    if with_count:
      def zone(i, _):
        ones_v[i, pl.ds(0, 16)] = zero16
        return 0
      lax.fori_loop(0, CH, zone, 0)
      for j in range(zchunks):
        pltpu.sync_copy(ones_v, cnt_sh.at[pl.ds((s * zchunks + j) * CH, CH)])
      def frow(i, _):
        ones_v[i, pl.ds(0, 16)] = one16
        return 0
      lax.fori_loop(0, CH, frow, 0)

    pltpu.sync_copy(srcidx.at[wid], src_slab)
    pltpu.sync_copy(dstidx.at[wid], dst_slab)
    plsc.subcore_barrier()

    def gather(j, buf, sem):
      pltpu.async_copy(feat.at[src_slab.at[j]], buf, sem)

    def scatter(j, buf):
      pltpu.sync_copy(buf, acc_sh.at[dst_slab.at[j]], add=True)
      if with_count:
        pltpu.sync_copy(ones_v, cnt_sh.at[dst_slab.at[j]], add=True)

    gather(0, rows0, sem0)

    def step(j, _):
      pltpu.async_copy(feat.at[src_slab.at[2 * j + 1]], rows1, sem1)
      pltpu.make_async_copy(feat.at[src_slab.at[2 * j]], rows0, sem0).wait()
      scatter(2 * j, rows0)
      @pl.when(2 * j + 2 < CPW)
      def _():
        pltpu.async_copy(feat.at[src_slab.at[2 * j + 2]], rows0, sem0)
      pltpu.make_async_copy(feat.at[src_slab.at[2 * j + 1]], rows1, sem1).wait()
      scatter(2 * j + 1, rows1)
      return 0

    lax.fori_loop(0, CPW // 2, step, 0)
    if CPW % 2:
      pltpu.make_async_copy(feat.at[src_slab.at[CPW - 1]], rows0, sem0).wait()
      scatter(CPW - 1, rows0)

    plsc.subcore_barrier()
    r0 = s * rows_per_tile
    pltpu.sync_copy(acc_sh.at[pl.ds(r0, rows_per_tile)],
                    out_acc.at[c, pl.ds(r0, rows_per_tile)])
    if with_count:
      pltpu.sync_copy(cnt_sh.at[pl.ds(r0, rows_per_tile)],
                      out_cnt.at[c, pl.ds(r0, rows_per_tile)])

  return pl.kernel(
      body, out_type=out_type, mesh=mesh, scratch_types=scratch,
      compiler_params=pltpu.CompilerParams(use_tc_tiling_on_sc=False))


def _pre_body(x_ref, wl_ref, wr_ref, p_ref, r_ref):
  xb = x_ref[...]
  p_ref[...] = jnp.dot(xb, wl_ref[...], preferred_element_type=jnp.float32)
  r_ref[...] = jnp.dot(xb, wr_ref[...], preferred_element_type=jnp.float32)


def _mid_body(acc_ref, cnt_ref, r_ref, b_ref, h_ref):
  a = acc_ref[0] + acc_ref[1]
  cnt = cnt_ref[0, :, 0:1] + cnt_ref[1, :, 0:1]
  mean = a / jnp.maximum(cnt, 1.0)
  h_ref[...] = jnp.maximum(mean + b_ref[...] + r_ref[...], 0.0)


def _post_body(acc_ref, cnt_ref, h_ref, wl_ref, wr_ref, b_ref, o_ref):
  a = acc_ref[0] + acc_ref[1]
  cnt = cnt_ref[0, :, 0:1] + cnt_ref[1, :, 0:1]
  mean = a / jnp.maximum(cnt, 1.0)
  hb = h_ref[...]
  z = (jnp.dot(mean, wl_ref[...], preferred_element_type=jnp.float32)
       + jnp.dot(hb, wr_ref[...], preferred_element_type=jnp.float32)
       + b_ref[...])
  m = jnp.max(z, axis=1, keepdims=True)
  lse = jnp.log(jnp.sum(jnp.exp(z - m), axis=1, keepdims=True))
  o_ref[...] = z - m - lse


@jax.jit
def kernel(x, edge_index, Wl1, bl1, Wr1, Wl2, bl2, Wr2):
  N, F = x.shape
  H = Wl1.shape[0]
  C = Wl2.shape[0]
  E = edge_index.shape[1]

  CPW = -(-E // (NW * CH))
  EP = NW * CPW * CH
  NP = -(-(N + 1) // (NS * CH)) * (NS * CH)

  src = jnp.concatenate(
      [edge_index[0], jnp.zeros((EP - E,), jnp.int32)]).reshape(NW, CPW, CH)
  dst = jnp.concatenate(
      [edge_index[1], jnp.full((EP - E,), N, jnp.int32)]).reshape(NW, CPW, CH)

  RB = 1000
  grid = -(-N // RB)

  p1, r1 = pl.pallas_call(
      _pre_body,
      grid=(grid,),
      in_specs=[
          pl.BlockSpec((RB, F), lambda i: (i, 0)),
          pl.BlockSpec((F, H), lambda i: (0, 0)),
          pl.BlockSpec((F, H), lambda i: (0, 0)),
      ],
      out_specs=[
          pl.BlockSpec((RB, H), lambda i: (i, 0)),
          pl.BlockSpec((RB, H), lambda i: (i, 0)),
      ],
      out_shape=[
          jax.ShapeDtypeStruct((N, H), jnp.float32),
          jax.ShapeDtypeStruct((N, H), jnp.float32),
      ],
  )(x, Wl1.T, Wr1.T)

  acc1, cntacc = _make_sc_segment_sum(NP, CPW, H, True)(p1, src, dst)

  h = pl.pallas_call(
      _mid_body,
      grid=(grid,),
      in_specs=[
          pl.BlockSpec((NC, RB, H), lambda i: (0, i, 0)),
          pl.BlockSpec((NC, RB, 16), lambda i: (0, i, 0)),
          pl.BlockSpec((RB, H), lambda i: (i, 0)),
          pl.BlockSpec((1, H), lambda i: (0, 0)),
      ],
      out_specs=pl.BlockSpec((RB, H), lambda i: (i, 0)),
      out_shape=jax.ShapeDtypeStruct((N, H), jnp.float32),
  )(acc1, cntacc, r1, bl1.reshape(1, H))

  (acc2,) = _make_sc_segment_sum(NP, CPW, H, False)(h, src, dst)

  out = pl.pallas_call(
      _post_body,
      grid=(grid,),
      in_specs=[
          pl.BlockSpec((NC, RB, H), lambda i: (0, i, 0)),
          pl.BlockSpec((NC, RB, 16), lambda i: (0, i, 0)),
          pl.BlockSpec((RB, H), lambda i: (i, 0)),
          pl.BlockSpec((H, C), lambda i: (0, 0)),
          pl.BlockSpec((H, C), lambda i: (0, 0)),
          pl.BlockSpec((1, C), lambda i: (0, 0)),
      ],
      out_specs=pl.BlockSpec((RB, C), lambda i: (i, 0)),
      out_shape=jax.ShapeDtypeStruct((N, C), jnp.float32),
  )(acc2, cntacc, h, Wl2.T, Wr2.T, bl2.reshape(1, C))

  return out
